# baseline (speedup 1.0000x reference)
import jax
import jax.numpy as jnp
from jax.experimental import pallas as pl
from jax.experimental.pallas import tpu as pltpu

_K = 1_000_000
_D = 64
_Q = 16
_GRP = 8
_W = _GRP * _D
_S = 5
_BR = 1000
_KR = _K // _GRP
_NB = _KR // _BR
_G = _NB // _S


def _nn_kernel(b1_ref, qnt_ref, *refs):
    c_refs = refs[:_S]
    dist_ref, idx_ref = refs[_S], refs[_S + 1]
    i = pl.program_id(0)

    @pl.when(i == 0)
    def _init():
        dist_ref[...] = jnp.full_like(dist_ref, jnp.inf)
        idx_ref[...] = jnp.zeros_like(idx_ref)

    b1 = b1_ref[...]
    qnt = qnt_ref[...]
    row2 = jax.lax.broadcasted_iota(jnp.int32, (_W, _GRP * _Q), 0)
    col2 = jax.lax.broadcasted_iota(jnp.int32, (_W, _GRP * _Q), 1)
    ones_b = (row2 // _D == col2 // _Q).astype(jnp.float32)
    lane = jax.lax.broadcasted_iota(jnp.int32, (1, _GRP * _Q), 1)
    g_of_lane = lane // _Q

    for s in range(_S):
        c = c_refs[s][...]
        m1 = jnp.dot(c, b1, preferred_element_type=jnp.float32)
        m2 = jnp.dot(c * c, ones_b, preferred_element_type=jnp.float32)
        d2 = (m1 + m2) + qnt
        lmin = jnp.min(d2, axis=0, keepdims=True)
        lrow = jnp.argmin(d2, axis=0).astype(jnp.int32)[None, :]
        gidx = _GRP * ((i * _S + s) * _BR + lrow) + g_of_lane
        better = lmin < dist_ref[...]
        dist_ref[...] = jnp.where(better, lmin, dist_ref[...])
        idx_ref[...] = jnp.where(better, gidx, idx_ref[...])

    @pl.when(i == _G - 1)
    def _finish():
        bd = dist_ref[0:1, 0:_Q]
        bi = idx_ref[0:1, 0:_Q]
        for g in range(1, _GRP):
            vd = dist_ref[0:1, g * _Q:(g + 1) * _Q]
            vi = idx_ref[0:1, g * _Q:(g + 1) * _Q]
            upd = vd < bd
            bd = jnp.where(upd, vd, bd)
            bi = jnp.where(upd, vi, bi)
        dist_ref[0:1, 0:_Q] = jnp.sqrt(jnp.maximum(bd, 0.0))
        idx_ref[0:1, 0:_Q] = bi


def kernel(face_embedding, centroids):
    qt = face_embedding.T
    b1 = -2.0 * jnp.kron(jnp.eye(_GRP, dtype=jnp.float32), qt)
    qn = jnp.sum(face_embedding * face_embedding, axis=1)
    qnt = jnp.tile(qn, _GRP)[None, :]
    cpacked = centroids.reshape(_KR, _W)
    in_specs = [
        pl.BlockSpec((_W, _GRP * _Q), lambda i: (0, 0)),
        pl.BlockSpec((1, _GRP * _Q), lambda i: (0, 0)),
    ]
    for s in range(_S):
        in_specs.append(
            pl.BlockSpec((_BR, _W), lambda i, s=s: (i * _S + s, 0)))
    dist, idx = pl.pallas_call(
        _nn_kernel,
        grid=(_G,),
        in_specs=in_specs,
        out_specs=[
            pl.BlockSpec((1, _GRP * _Q), lambda i: (0, 0)),
            pl.BlockSpec((1, _GRP * _Q), lambda i: (0, 0)),
        ],
        out_shape=[
            jax.ShapeDtypeStruct((1, _GRP * _Q), jnp.float32),
            jax.ShapeDtypeStruct((1, _GRP * _Q), jnp.int32),
        ],
    )(b1, qnt, *([cpacked] * _S))
    return dist[0, :_Q], idx[0, :_Q]
